# Initial kernel scaffold; baseline (speedup 1.0000x reference)
import jax
import jax.numpy as jnp
from jax.experimental import pallas as pl


def kernel(x):
    raise NotImplementedError("write your pallas kernel here")


# speedup vs baseline: 1.5118x; 1.5118x over previous
import jax
import jax.numpy as jnp
from jax.experimental import pallas as pl

_CHANNELS = 32768
_ROWS = 128
_BLOCK_ROWS = 16


def _onehot_argmax_kernel(x_ref, o_ref):
    x = x_ref[...]
    m = jnp.max(x, axis=1, keepdims=True)
    col = jax.lax.broadcasted_iota(jnp.int32, x.shape, 1)
    idx = jnp.min(jnp.where(x == m, col, _CHANNELS), axis=1, keepdims=True)
    o_ref[...] = (col == idx).astype(jnp.float32)


def kernel(x):
    return pl.pallas_call(
        _onehot_argmax_kernel,
        grid=(_ROWS // _BLOCK_ROWS,),
        in_specs=[pl.BlockSpec((_BLOCK_ROWS, _CHANNELS), lambda i: (i, 0))],
        out_specs=pl.BlockSpec((_BLOCK_ROWS, _CHANNELS), lambda i: (i, 0)),
        out_shape=jax.ShapeDtypeStruct((_ROWS, _CHANNELS), jnp.float32),
    )(x)


# speedup vs baseline: 1.6821x; 1.1127x over previous
import jax
import jax.numpy as jnp
from jax.experimental import pallas as pl

_CHANNELS = 32768
_ROWS = 128
_BLOCK_ROWS = 32


def _onehot_argmax_kernel(x_ref, o_ref):
    x = x_ref[...]
    m = jnp.max(x, axis=1, keepdims=True)
    col = jax.lax.broadcasted_iota(jnp.int32, x.shape, 1)
    idx = jnp.min(jnp.where(x == m, col, _CHANNELS), axis=1, keepdims=True)
    o_ref[...] = (col == idx).astype(jnp.float32)


def kernel(x):
    return pl.pallas_call(
        _onehot_argmax_kernel,
        grid=(_ROWS // _BLOCK_ROWS,),
        in_specs=[pl.BlockSpec((_BLOCK_ROWS, _CHANNELS), lambda i: (i, 0))],
        out_specs=pl.BlockSpec((_BLOCK_ROWS, _CHANNELS), lambda i: (i, 0)),
        out_shape=jax.ShapeDtypeStruct((_ROWS, _CHANNELS), jnp.float32),
    )(x)


# speedup vs baseline: 1.9704x; 1.1713x over previous
import jax
import jax.numpy as jnp
from jax.experimental import pallas as pl

_CHANNELS = 32768
_ROWS = 128
_BLOCK_ROWS = 64


def _onehot_argmax_kernel(x_ref, o_ref):
    x = x_ref[...]
    m = jnp.max(x, axis=1, keepdims=True)
    col = jax.lax.broadcasted_iota(jnp.int32, x.shape, 1)
    idx = jnp.min(jnp.where(x == m, col, _CHANNELS), axis=1, keepdims=True)
    o_ref[...] = (col == idx).astype(jnp.float32)


def kernel(x):
    return pl.pallas_call(
        _onehot_argmax_kernel,
        grid=(_ROWS // _BLOCK_ROWS,),
        in_specs=[pl.BlockSpec((_BLOCK_ROWS, _CHANNELS), lambda i: (i, 0))],
        out_specs=pl.BlockSpec((_BLOCK_ROWS, _CHANNELS), lambda i: (i, 0)),
        out_shape=jax.ShapeDtypeStruct((_ROWS, _CHANNELS), jnp.float32),
    )(x)
